# baseline (speedup 1.0000x reference)
import jax
import jax.numpy as jnp
from jax import lax
from jax.experimental import pallas as pl
from jax.experimental.pallas import tpu as pltpu
from jax.experimental.pallas import tpu_sc as plsc

_NUMEL = 16777216
_NNZ = 1677721
_LR = 0.1

_NC = 2
_NS = 16
_G = 2048
_TOTAL = 1703936
_SCAN = _TOTAL // _NS
_NCH = _SCAN // _G
_SBUF = 4

_SEG = 983040
_TRASH = _SEG
_NPASS = 9
_LASTSEG = (_NUMEL - 2 * _SEG * (_NPASS - 1)) // 2
_D = 4096
_FULL_KCH = _SEG // (_NS * _D)
_LAST_KCH = _LASTSEG // (_NS * _D)


def _sc_body(idx_hbm, val_hbm, param_hbm, out_hbm, seg, zbuf, *scr):
    idx_v = scr[0:_SBUF]
    val_v = scr[_SBUF:2 * _SBUF]
    da = scr[2 * _SBUF:2 * _SBUF + 2]
    db = scr[2 * _SBUF + 2:2 * _SBUF + 4]
    n0 = 2 * _SBUF + 4
    sem_ld = scr[n0:n0 + _SBUF]
    sem_s = scr[n0 + _SBUF:n0 + 2 * _SBUF]
    sem_z = scr[n0 + 2 * _SBUF]
    sem_da = scr[n0 + 2 * _SBUF + 1:n0 + 2 * _SBUF + 3]
    sem_db = scr[n0 + 2 * _SBUF + 3:n0 + 2 * _SBUF + 5]
    sem_do = scr[n0 + 2 * _SBUF + 5:n0 + 2 * _SBUF + 7]
    c = lax.axis_index("c")
    s = lax.axis_index("s")
    scan0 = s * _SCAN

    @pl.loop(0, _D // 16, unroll=4)
    def _z(i):
        zbuf[pl.ds(i * 16, 16)] = jnp.zeros((16,), jnp.float32)

    def zero_chunk(off, words):
        pltpu.make_async_copy(zbuf.at[pl.ds(0, words)],
                              seg.at[pl.ds(off, words)], sem_z).start()

    def zero_wait(words):
        pltpu.make_async_copy(zbuf.at[pl.ds(0, words)],
                              seg.at[pl.ds(0, words)], sem_z).wait()

    @pl.loop(0, _NPASS)
    def _pass(p):
        is_last = p == _NPASS - 1
        size = jnp.where(is_last, _LASTSEG, _SEG)
        pbase = 2 * p * _SEG + c * size
        kfull = jnp.where(is_last, _LAST_KCH, _FULL_KCH)
        tile0 = s * kfull * _D

        @pl.loop(0, kfull)
        def _zi(k):
            zero_chunk(tile0 + k * _D, _D)

        @pl.loop(0, kfull)
        def _zw(k):
            zero_wait(_D)

        plsc.subcore_barrier()

        def start_scan(t, j):
            off = scan0 + t * _G
            pltpu.make_async_copy(idx_hbm.at[pl.ds(off, _G)], idx_v[j],
                                  sem_ld[j]).start()
            pltpu.make_async_copy(val_hbm.at[pl.ds(off, _G)], val_v[j],
                                  sem_ld[j]).start()

        for j in range(_SBUF):
            start_scan(j, j)

        usize = size.astype(jnp.uint32)

        @pl.loop(0, _NCH, step=_SBUF)
        def _scan(t0):
            for j in range(_SBUF):
                pltpu.make_async_copy(idx_hbm.at[pl.ds(0, _G)], idx_v[j],
                                      sem_ld[j]).wait()
                pltpu.make_async_copy(val_hbm.at[pl.ds(0, _G)], val_v[j],
                                      sem_ld[j]).wait()

                @pl.loop(0, _G // 16, unroll=4)
                def _cmp(i, j=j):
                    sl = pl.ds(i * 16, 16)
                    loc = idx_v[j][sl] - pbase
                    ok = plsc.bitcast(loc, jnp.uint32) < usize
                    idx_v[j][sl] = jnp.where(ok, loc, _TRASH)

                pltpu.make_async_copy(val_v[j], seg.at[idx_v[j]],
                                      sem_s[j]).start()
            for j in range(_SBUF):
                pltpu.make_async_copy(val_v[j], seg.at[idx_v[j]],
                                      sem_s[j]).wait()

                @pl.when(t0 + _SBUF + j < _NCH)
                def _pf(t0=t0, j=j):
                    start_scan(t0 + _SBUF + j, j)

        plsc.subcore_barrier()

        def drain_chunk(soff, b):
            goff = pbase + soff
            pltpu.make_async_copy(seg.at[pl.ds(soff, _D)],
                                  da[b], sem_da[b]).start()
            pltpu.make_async_copy(param_hbm.at[pl.ds(goff, _D)],
                                  db[b], sem_db[b]).start()
            pltpu.make_async_copy(seg.at[pl.ds(soff, _D)],
                                  da[b], sem_da[b]).wait()
            pltpu.make_async_copy(param_hbm.at[pl.ds(goff, _D)],
                                  db[b], sem_db[b]).wait()

            @pl.loop(0, _D // 16, unroll=4)
            def _add(i, b=b):
                sl = pl.ds(i * 16, 16)
                db[b][sl] = db[b][sl] + _LR * da[b][sl]

            pltpu.make_async_copy(db[b], out_hbm.at[pl.ds(goff, _D)],
                                  sem_do[b]).start()

        def drain_wait(b):
            pltpu.make_async_copy(db[b], out_hbm.at[pl.ds(0, _D)],
                                  sem_do[b]).wait()

        @pl.loop(0, kfull, step=2)
        def _drain(k):
            @pl.when(k >= 2)
            def _w0():
                drain_wait(0)

            drain_chunk(tile0 + k * _D, 0)

            @pl.when(k + 1 < kfull)
            def _c1():
                @pl.when(k >= 1)
                def _w1():
                    drain_wait(1)

                drain_chunk(tile0 + (k + 1) * _D, 1)

        drain_wait(0)

        @pl.when(lax.rem(kfull, 2) == 0)
        def _w1f():
            drain_wait(1)

        plsc.subcore_barrier()


_sc_update = pl.kernel(
    _sc_body,
    out_type=jax.ShapeDtypeStruct((_NUMEL,), jnp.float32),
    mesh=plsc.VectorSubcoreMesh(core_axis_name="c", subcore_axis_name="s"),
    scratch_types=(
        [pltpu.VMEM_SHARED((_SEG + 16,), jnp.float32),
         pltpu.VMEM((_D,), jnp.float32)]
        + [pltpu.VMEM((_G,), jnp.int32) for _ in range(_SBUF)]
        + [pltpu.VMEM((_G,), jnp.float32) for _ in range(_SBUF)]
        + [pltpu.VMEM((_D,), jnp.float32) for _ in range(4)]
        + [pltpu.SemaphoreType.DMA] * (2 * _SBUF + 7)),
)


def kernel(param, values, indices):
    idx = indices.astype(jnp.int32)
    pad = _TOTAL - _NNZ
    idxp = jnp.pad(idx, (0, pad), mode="wrap")
    valp = jnp.pad(values, (0, pad), mode="wrap")
    return _sc_update(idxp, valp, param)


# speedup vs baseline: 4.9613x; 4.9613x over previous
import jax
import jax.numpy as jnp
from jax import lax
from jax.experimental import pallas as pl
from jax.experimental.pallas import tpu as pltpu
from jax.experimental.pallas import tpu_sc as plsc

_NUMEL = 16777216
_NNZ = 1677721
_LR = 0.1

_NC = 2
_NS = 16
_NW = _NC * _NS
_G = 4096
_GROUPS = 13
_NBUF = 5
_P = _G * _GROUPS
_TOTAL = _NW * _P


def _sc_body(idx_hbm, val_hbm, param_hbm, out_ref, *scr):
    idx_v = scr[0:_NBUF]
    val_v = scr[_NBUF:2 * _NBUF]
    gat_v = scr[2 * _NBUF:3 * _NBUF]
    sem_ld = scr[3 * _NBUF:4 * _NBUF]
    sem_g = scr[4 * _NBUF:5 * _NBUF]
    sem_s = scr[5 * _NBUF:6 * _NBUF]
    c = lax.axis_index("c")
    s = lax.axis_index("s")
    wid = s * _NC + c
    base0 = wid * _P

    def start_load(t, m):
        off = base0 + t * _G
        pltpu.make_async_copy(idx_hbm.at[pl.ds(off, _G)], idx_v[m],
                              sem_ld[m]).start()
        pltpu.make_async_copy(val_hbm.at[pl.ds(off, _G)], val_v[m],
                              sem_ld[m]).start()

    def wait_load(m):
        pltpu.make_async_copy(idx_hbm.at[pl.ds(0, _G)], idx_v[m],
                              sem_ld[m]).wait()
        pltpu.make_async_copy(val_hbm.at[pl.ds(0, _G)], val_v[m],
                              sem_ld[m]).wait()

    def fire_gather(m):
        pltpu.make_async_copy(param_hbm.at[idx_v[m]], gat_v[m],
                              sem_g[m]).start()

    def process(n):
        pltpu.make_async_copy(param_hbm.at[idx_v[n]], gat_v[n],
                              sem_g[n]).wait()

        @pl.loop(0, _G // 16, unroll=4)
        def _cmp(i):
            sl = pl.ds(i * 16, 16)
            gat_v[n][sl] = gat_v[n][sl] + _LR * val_v[n][sl]

        pltpu.make_async_copy(gat_v[n], out_ref.at[idx_v[n]],
                              sem_s[n]).start()

    def drain_scatter(m):
        pltpu.make_async_copy(gat_v[m], out_ref.at[idx_v[m]],
                              sem_s[m]).wait()

    for t in range(_GROUPS):
        m = t % _NBUF
        if t >= _NBUF:
            drain_scatter(m)
        start_load(t, m)
        if t >= 2:
            process((t - 2) % _NBUF)
        wait_load(m)
        fire_gather(m)
    for t in (_GROUPS - 2, _GROUPS - 1):
        process(t % _NBUF)
    for t in range(_GROUPS - _NBUF, _GROUPS):
        drain_scatter(t % _NBUF)


_sc_update = pl.kernel(
    _sc_body,
    out_type=(),
    mesh=plsc.VectorSubcoreMesh(core_axis_name="c", subcore_axis_name="s"),
    scratch_types=(
        [pltpu.VMEM((_G,), jnp.int32) for _ in range(_NBUF)]
        + [pltpu.VMEM((_G,), jnp.float32) for _ in range(_NBUF)]
        + [pltpu.VMEM((_G,), jnp.float32) for _ in range(_NBUF)]
        + [pltpu.SemaphoreType.DMA] * (3 * _NBUF)),
)


def kernel(param, values, indices):
    idx = indices.astype(jnp.int32)
    pad = _TOTAL - _NNZ
    idxp = jnp.pad(idx, (0, pad), mode="wrap")
    valp = jnp.pad(values, (0, pad), mode="wrap")
    out_ref = jax.new_ref(param)
    _sc_update(idxp, valp, param, out_ref)
    return out_ref[...]
